# Initial kernel scaffold; baseline (speedup 1.0000x reference)
import jax
import jax.numpy as jnp
from jax.experimental import pallas as pl


def kernel(x_lit, x_cls, edge_lit, edge_cls, enc_W1, enc_b1, enc_W2, enc_b2, lit_W, lit_b, cls_W, cls_b, out_W1, out_b1, out_W2, out_b2):
    raise NotImplementedError("write your pallas kernel here")


# speedup vs baseline: 6.0002x; 6.0002x over previous
import functools

import jax
import jax.numpy as jnp
from jax import lax
from jax.experimental import pallas as pl
from jax.experimental.pallas import tpu as pltpu
from jax.experimental.pallas import tpu_sc as plsc

_N_LIT = 10000
_N_CLS = 5000
_E = 320000
_C = 128
_OUT_DIM = 2

_NW = 32
_EPW = _E // _NW
_K = 80
_NCHUNK = _EPW // _K
_N_CLS_PAD = 5120
_N_LIT_PAD = 10112

_MESH = plsc.VectorSubcoreMesh(core_axis_name="c", subcore_axis_name="s")


def _make_seg_sum(n_pad):
    stripe = n_pad // 16

    @functools.partial(
        pl.kernel,
        out_type=jax.ShapeDtypeStruct((2, n_pad, _C), jnp.float32),
        mesh=_MESH,
        scratch_types=[
            pltpu.VMEM((_NCHUNK, _K), jnp.int32),
            pltpu.VMEM((_NCHUNK, _K), jnp.int32),
            pltpu.VMEM((_K, _C), jnp.float32),
            pltpu.MemorySpace.VMEM_SHARED((n_pad, _C), jnp.float32),
            pltpu.SemaphoreType.DMA,
        ],
    )
    def seg_sum(table, esrc, edst, zeros, out, idx_s, idx_d, rows, acc, sem):
        c = lax.axis_index("c")
        s = lax.axis_index("s")
        wid = s * 2 + c
        pltpu.sync_copy(zeros.at[pl.ds(s * stripe, stripe)],
                        acc.at[pl.ds(s * stripe, stripe)])
        pltpu.sync_copy(esrc.at[wid], idx_s)
        pltpu.sync_copy(edst.at[wid], idx_d)
        plsc.subcore_barrier()

        def body(j, carry):
            pltpu.async_copy(table.at[idx_s.at[j]], rows, sem).wait()
            pltpu.sync_copy(rows, acc.at[idx_d.at[j]], add=True)
            return carry

        lax.fori_loop(0, _NCHUNK, body, 0)
        plsc.subcore_barrier()
        pltpu.sync_copy(acc.at[pl.ds(s * stripe, stripe)],
                        out.at[c, pl.ds(s * stripe, stripe)])

    return seg_sum


_seg_to_cls = _make_seg_sum(_N_CLS_PAD)
_seg_to_lit = _make_seg_sum(_N_LIT_PAD)

def _make_count(n_pad):
    stripe = n_pad // 16

    @functools.partial(
        pl.kernel,
        out_type=jax.ShapeDtypeStruct((2, n_pad, _C), jnp.float32),
        mesh=_MESH,
        scratch_types=[
            pltpu.VMEM((_NCHUNK, _K), jnp.int32),
            pltpu.VMEM((_K, _C), jnp.float32),
            pltpu.MemorySpace.VMEM_SHARED((n_pad, _C), jnp.float32),
        ],
    )
    def count(edst, zeros, ones, out, idx_d, ones_v, acc):
        c = lax.axis_index("c")
        s = lax.axis_index("s")
        wid = s * 2 + c
        pltpu.sync_copy(zeros.at[pl.ds(s * stripe, stripe)],
                        acc.at[pl.ds(s * stripe, stripe)])
        pltpu.sync_copy(ones, ones_v)
        pltpu.sync_copy(edst.at[wid], idx_d)
        plsc.subcore_barrier()

        def body(j, carry):
            pltpu.sync_copy(ones_v, acc.at[idx_d.at[j]], add=True)
            return carry

        lax.fori_loop(0, _NCHUNK, body, 0)
        plsc.subcore_barrier()
        pltpu.sync_copy(acc.at[pl.ds(s * stripe, stripe)],
                        out.at[c, pl.ds(s * stripe, stripe)])

    return count


_cnt_cls_kernel = _make_count(_N_CLS_PAD)
_cnt_lit_kernel = _make_count(_N_LIT_PAD)


def _mlp2(x, w1, b1, w2, b2, blk):
    n, d1 = x.shape
    dh = w1.shape[1]
    do = w2.shape[1]

    def body(x_ref, w1_ref, b1_ref, w2_ref, b2_ref, o_ref):
        z = jnp.dot(x_ref[...], w1_ref[...],
                    preferred_element_type=jnp.float32) + b1_ref[...]
        h = z * jax.nn.sigmoid(z)
        o_ref[...] = jnp.dot(h, w2_ref[...],
                             preferred_element_type=jnp.float32) + b2_ref[...]

    return pl.pallas_call(
        body,
        grid=(n // blk,),
        in_specs=[
            pl.BlockSpec((blk, d1), lambda i: (i, 0)),
            pl.BlockSpec((d1, dh), lambda i: (0, 0)),
            pl.BlockSpec((1, dh), lambda i: (0, 0)),
            pl.BlockSpec((dh, do), lambda i: (0, 0)),
            pl.BlockSpec((1, do), lambda i: (0, 0)),
        ],
        out_specs=pl.BlockSpec((blk, do), lambda i: (i, 0)),
        out_shape=jax.ShapeDtypeStruct((n, do), jnp.float32),
    )(x, w1, b1.reshape(1, dh), w2, b2.reshape(1, do))


def _layer_update(h, partials, cnts, w, b, blk):
    n = h.shape[0]

    def body(h_ref, p_ref, c_ref, w_ref, b_ref, o_ref):
        hx = h_ref[...]
        cnt = c_ref[0] + c_ref[1]
        agg = (p_ref[0] + p_ref[1]) / jnp.maximum(cnt, 1.0)
        z = (jnp.dot(hx, w_ref[0], preferred_element_type=jnp.float32)
             + jnp.dot(agg, w_ref[1], preferred_element_type=jnp.float32)
             + b_ref[...])
        o_ref[...] = hx + z * jax.nn.sigmoid(z)

    return pl.pallas_call(
        body,
        grid=(n // blk,),
        in_specs=[
            pl.BlockSpec((blk, _C), lambda i: (i, 0)),
            pl.BlockSpec((2, blk, _C), lambda i: (0, i, 0)),
            pl.BlockSpec((2, blk, _C), lambda i: (0, i, 0)),
            pl.BlockSpec((2, _C, _C), lambda i: (0, 0, 0)),
            pl.BlockSpec((1, _C), lambda i: (0, 0)),
        ],
        out_specs=pl.BlockSpec((blk, _C), lambda i: (i, 0)),
        out_shape=jax.ShapeDtypeStruct((n, _C), jnp.float32),
    )(h, partials, cnts, w, b.reshape(1, _C))


def kernel(x_lit, x_cls, edge_lit, edge_cls, enc_W1, enc_b1, enc_W2, enc_b2,
           lit_W, lit_b, cls_W, cls_b, out_W1, out_b1, out_W2, out_b2):
    el = edge_lit.reshape(_NW, _NCHUNK, _K)
    ec = edge_cls.reshape(_NW, _NCHUNK, _K)
    zeros128 = jnp.zeros((_N_LIT_PAD, _C), jnp.float32)
    ones128 = jnp.ones((_K, _C), jnp.float32)

    cnt_cls = _cnt_cls_kernel(ec, zeros128, ones128)
    cnt_lit = _cnt_lit_kernel(el, zeros128, ones128)

    x_all = jnp.concatenate([x_lit, x_cls], axis=0)
    h_all = _mlp2(x_all, enc_W1, enc_b1, enc_W2, enc_b2, blk=1000)
    h_lit, h_cls = h_all[:_N_LIT], h_all[_N_LIT:]

    n_layers = lit_W.shape[0]
    cls_W2 = cls_W.reshape(n_layers, 2, _C, _C)
    lit_W2 = lit_W.reshape(n_layers, 2, _C, _C)

    for l in range(n_layers):
        p_cls = _seg_to_cls(h_lit, el, ec, zeros128)
        h_cls = _layer_update(h_cls, p_cls, cnt_cls, cls_W2[l], cls_b[l], blk=1000)
        p_lit = _seg_to_lit(h_cls, ec, el, zeros128)
        h_lit = _layer_update(h_lit, p_lit, cnt_lit, lit_W2[l], lit_b[l], blk=1000)

    hv = h_lit.reshape(_N_CLS, 2 * _C)
    w2p = jnp.zeros((2 * _C, _C), jnp.float32).at[:, :_OUT_DIM].set(out_W2)
    b2p = jnp.zeros((_C,), jnp.float32).at[:_OUT_DIM].set(out_b2)
    y = _mlp2(hv, out_W1, out_b1, w2p, b2p, blk=1000)
    return y[:, :_OUT_DIM]
